# baseline (speedup 1.0000x reference)
import jax
import jax.numpy as jnp
from jax.experimental import pallas as pl
from jax.experimental.pallas import tpu as pltpu

_NS = 2


def _netvlad_kernel(*refs):
    x_refs = refs[:_NS]
    w_ref, c_ref, o_ref = refs[_NS:]
    K, C = w_ref.shape
    Cs = C // _NS
    x_bf = [x_refs[j][0].astype(jnp.bfloat16) for j in range(_NS)]
    logits = jnp.dot(w_ref[:, 0:Cs], x_bf[0],
                     preferred_element_type=jnp.float32)
    for j in range(1, _NS):
        logits = logits + jnp.dot(w_ref[:, j * Cs:(j + 1) * Cs],
                                  x_bf[j],
                                  preferred_element_type=jnp.float32)
    m = jnp.max(logits, axis=0, keepdims=True)
    e = jnp.exp(logits - m)
    s = jnp.sum(e, axis=0, keepdims=True)
    a = e / s
    a_sum = jnp.sum(a, axis=1, keepdims=True)
    a_bf = a.astype(jnp.bfloat16)
    vlads = []
    sq = 0.0
    for j in range(_NS):
        ax = jax.lax.dot_general(
            a_bf, x_bf[j], (((1,), (1,)), ((), ())),
            preferred_element_type=jnp.float32)
        vlad = ax - a_sum * c_ref[:, j * Cs:(j + 1) * Cs]
        vlads.append(vlad)
        sq = sq + jnp.sum(vlad * vlad)
    inv = 1.0 / jnp.maximum(jnp.sqrt(sq), 1e-12)
    for j in range(_NS):
        o_ref[0, :, j * Cs:(j + 1) * Cs] = vlads[j] * inv


def kernel(x, conv_w, centroids):
    B, C, N = x.shape
    K = conv_w.shape[0]
    Cs = C // _NS
    x_specs = [
        pl.BlockSpec((1, Cs, N), lambda b, j=j: (b, j, 0)) for j in range(_NS)
    ]
    out = pl.pallas_call(
        _netvlad_kernel,
        grid=(B,),
        in_specs=x_specs + [
            pl.BlockSpec((K, C), lambda b: (0, 0)),
            pl.BlockSpec((K, C), lambda b: (0, 0)),
        ],
        out_specs=pl.BlockSpec((1, K, C), lambda b: (b, 0, 0)),
        out_shape=jax.ShapeDtypeStruct((B, K, C), jnp.float32),
        compiler_params=pltpu.CompilerParams(
            dimension_semantics=("arbitrary",),
        ),
    )(*([x] * _NS), conv_w.astype(jnp.bfloat16), centroids)
    return out.reshape(B, K * C)


# speedup vs baseline: 1.0242x; 1.0242x over previous
import jax
import jax.numpy as jnp
from jax.experimental import pallas as pl
from jax.experimental.pallas import tpu as pltpu

_NS = 2


def _netvlad_kernel(*refs):
    x_refs = refs[:_NS]
    w_ref, c_ref, o_ref = refs[_NS:]
    K, C = w_ref.shape
    Cs = C // _NS
    x_bf = [x_refs[j][0].astype(jnp.bfloat16) for j in range(_NS)]
    w_bf = w_ref[...].astype(jnp.bfloat16)
    logits = jnp.dot(w_bf[:, 0:Cs], x_bf[0],
                     preferred_element_type=jnp.float32)
    for j in range(1, _NS):
        logits = logits + jnp.dot(w_bf[:, j * Cs:(j + 1) * Cs],
                                  x_bf[j],
                                  preferred_element_type=jnp.float32)
    m = jnp.max(logits, axis=0, keepdims=True)
    e = jnp.exp(logits - m)
    s = jnp.sum(e, axis=0, keepdims=True)
    a = e / s
    a_sum = jnp.sum(a, axis=1, keepdims=True)
    a_bf = a.astype(jnp.bfloat16)
    vlads = []
    sq = 0.0
    for j in range(_NS):
        ax = jax.lax.dot_general(
            a_bf, x_bf[j], (((1,), (1,)), ((), ())),
            preferred_element_type=jnp.float32)
        vlad = ax - a_sum * c_ref[:, j * Cs:(j + 1) * Cs]
        vlads.append(vlad)
        sq = sq + jnp.sum(vlad * vlad)
    inv = 1.0 / jnp.maximum(jnp.sqrt(sq), 1e-12)
    for j in range(_NS):
        o_ref[0, :, j * Cs:(j + 1) * Cs] = vlads[j] * inv


def kernel(x, conv_w, centroids):
    B, C, N = x.shape
    K = conv_w.shape[0]
    Cs = C // _NS
    x_specs = [
        pl.BlockSpec((1, Cs, N), lambda b, j=j: (b, j, 0)) for j in range(_NS)
    ]
    out = pl.pallas_call(
        _netvlad_kernel,
        grid=(B,),
        in_specs=x_specs + [
            pl.BlockSpec((K, C), lambda b: (0, 0)),
            pl.BlockSpec((K, C), lambda b: (0, 0)),
        ],
        out_specs=pl.BlockSpec((1, K, C), lambda b: (b, 0, 0)),
        out_shape=jax.ShapeDtypeStruct((B, K, C), jnp.float32),
        compiler_params=pltpu.CompilerParams(
            dimension_semantics=("arbitrary",),
        ),
    )(*([x] * _NS), conv_w, centroids)
    return out.reshape(B, K * C)
